# baseline (speedup 1.0000x reference)
import functools

import jax
import jax.numpy as jnp
from jax import lax
from jax.experimental import pallas as pl
from jax.experimental.pallas import tpu as pltpu
from jax.experimental.pallas import tpu_sc as plsc

B = 4096
S = 200
D = 64
H = 64
NCLS = 4
NCLS_PAD = 8
VOCAB = 1000000

NC = 2
NS = 16
NW = NC * NS
L = 16

B_PER_W = B // NW
C0, C1 = 104, 96
NBUF = 2
LANE_CHUNKS = D // L
ACC_UNROLL = 8
N_IDX = B_PER_W * S

_mesh = plsc.VectorSubcoreMesh(core_axis_name="c", subcore_axis_name="s")


@functools.partial(
    pl.kernel,
    mesh=_mesh,
    out_type=jax.ShapeDtypeStruct((B, D), jnp.float32),
    scratch_types=[
        pltpu.VMEM((N_IDX + L,), jnp.int32),
        pltpu.VMEM((N_IDX,), jnp.int32),
        *[pltpu.VMEM((S, 2 * D), jnp.float32) for _ in range(NBUF)],
        pltpu.VMEM((B_PER_W, D), jnp.float32),
        *[pltpu.SemaphoreType.DMA for _ in range(NBUF)],
    ],
)
def _pool_sums(x_hbm, emb2_hbm, out_hbm, idx_v, pair_v, *rest):
    bufs = rest[:NBUF]
    out_v = rest[NBUF]
    sems = rest[NBUF + 1 : NBUF + 1 + NBUF]

    wid = lax.axis_index("s") * NC + lax.axis_index("c")
    base = wid * B_PER_W

    pltpu.sync_copy(x_hbm.at[pl.ds(base * S, N_IDX)], idx_v.at[pl.ds(0, N_IDX)])

    def shift_body(i, _):
        pair_v[pl.ds(i * L, L)] = lax.shift_right_logical(
            idx_v[pl.ds(i * L, L)], 1)
        return ()

    lax.fori_loop(0, N_IDX // L, shift_body, ())

    def issue(row, buf, sem):
        pltpu.async_copy(emb2_hbm.at[pair_v.at[pl.ds(row * S, C0)]],
                         buf.at[pl.ds(0, C0)], sem)
        pltpu.async_copy(emb2_hbm.at[pair_v.at[pl.ds(row * S + C0, C1)]],
                         buf.at[pl.ds(C0, C1)], sem)

    def wait(row, buf, sem):
        pltpu.make_async_copy(emb2_hbm.at[pair_v.at[pl.ds(row * S, C0)]],
                              buf.at[pl.ds(0, C0)], sem).wait()
        pltpu.make_async_copy(emb2_hbm.at[pair_v.at[pl.ds(row * S + C0, C1)]],
                              buf.at[pl.ds(C0, C1)], sem).wait()

    def reduce_row(row, buf):
        def acc_body(j8, accs):
            new = list(accs)
            par = (idx_v[pl.ds(row * S + j8 * ACC_UNROLL, L)] & 1) * D
            for jj in range(ACC_UNROLL):
                j = j8 * ACC_UNROLL + jj
                half = par[jj]
                for k in range(LANE_CHUNKS):
                    new[k] = new[k] + buf[j, pl.ds(half + k * L, L)]
            return tuple(new)

        accs = tuple(jnp.zeros((L,), jnp.float32) for _ in range(LANE_CHUNKS))
        accs = lax.fori_loop(0, S // ACC_UNROLL, acc_body, accs)
        for k in range(LANE_CHUNKS):
            out_v[row, pl.ds(k * L, L)] = accs[k]

    for p in range(NBUF):
        issue(p, bufs[p], sems[p])

    def body(blk, _):
        for p in range(NBUF):
            row = blk * NBUF + p
            wait(row, bufs[p], sems[p])
            reduce_row(row, bufs[p])
            issue(row + NBUF, bufs[p], sems[p])
        return ()

    lax.fori_loop(0, B_PER_W // NBUF - 1, body, ())

    for p in range(NBUF):
        row = B_PER_W - NBUF + p
        wait(row, bufs[p], sems[p])
        reduce_row(row, bufs[p])

    pltpu.sync_copy(out_v, out_hbm.at[pl.ds(base, B_PER_W)])


def _mlp_body(sums_ref, w1t_ref, b1_ref, w2t_ref, b2_ref, out_ref):
    pooled = sums_ref[...] * (1.0 / S)
    h = jnp.dot(pooled, w1t_ref[...], preferred_element_type=jnp.float32)
    h = jnp.maximum(h + b1_ref[...], 0.0)
    out_ref[...] = (
        jnp.dot(h, w2t_ref[...], preferred_element_type=jnp.float32)
        + b2_ref[...]
    )


_mlp = pl.pallas_call(
    _mlp_body,
    out_shape=jax.ShapeDtypeStruct((B, NCLS_PAD), jnp.float32),
)


def kernel(x, emb, W1, b1, W2, b2):
    x32 = x.astype(jnp.int32).reshape(-1)
    emb2 = emb.reshape(VOCAB // 2, 2 * D)
    sums = _pool_sums(x32, emb2)
    w2t_pad = jnp.zeros((H, NCLS_PAD), jnp.float32).at[:, :NCLS].set(W2.T)
    b2_pad = jnp.zeros((1, NCLS_PAD), jnp.float32).at[0, :NCLS].set(b2)
    logits = _mlp(sums, W1.T, b1.reshape(1, H), w2t_pad, b2_pad)
    return logits[:, :NCLS]


# speedup vs baseline: 1.1776x; 1.1776x over previous
import functools

import jax
import jax.numpy as jnp
from jax import lax
from jax.experimental import pallas as pl
from jax.experimental.pallas import tpu as pltpu
from jax.experimental.pallas import tpu_sc as plsc

B = 4096
S = 200
D = 64
H = 64
NCLS = 4
NCLS_PAD = 8

NC = 2
NS = 16
NW = NC * NS
L = 16

B_PER_W = B // NW
C0, C1 = 104, 96
NBUF = 4
LANE_CHUNKS = D // L
ACC_UNROLL = 8
SP = 256
N_IDX = B_PER_W * SP

_mesh = plsc.VectorSubcoreMesh(core_axis_name="c", subcore_axis_name="s")


@functools.partial(
    pl.kernel,
    mesh=_mesh,
    compiler_params=pltpu.CompilerParams(use_tc_tiling_on_sc=False),
    out_type=jax.ShapeDtypeStruct((B, D), jnp.float32),
    scratch_types=[
        pltpu.VMEM((N_IDX,), jnp.int32),
        *[pltpu.VMEM((S, D), jnp.float32) for _ in range(NBUF)],
        pltpu.VMEM((B_PER_W, D), jnp.float32),
        *[pltpu.SemaphoreType.DMA for _ in range(NBUF)],
    ],
)
def _pool_sums(x_hbm, emb_hbm, out_hbm, idx_v, *rest):
    bufs = rest[:NBUF]
    out_v = rest[NBUF]
    sems = rest[NBUF + 1 : NBUF + 1 + NBUF]

    wid = lax.axis_index("s") * NC + lax.axis_index("c")
    base = wid * B_PER_W

    pltpu.sync_copy(x_hbm.at[pl.ds(base * SP, N_IDX)], idx_v)

    def issue(row, buf, sem):
        pltpu.async_copy(emb_hbm.at[idx_v.at[pl.ds(row * SP, C0)]],
                         buf.at[pl.ds(0, C0)], sem)
        pltpu.async_copy(emb_hbm.at[idx_v.at[pl.ds(row * SP + C0, C1)]],
                         buf.at[pl.ds(C0, C1)], sem)

    def wait(row, buf, sem):
        pltpu.make_async_copy(emb_hbm.at[idx_v.at[pl.ds(row * SP, C0)]],
                              buf.at[pl.ds(0, C0)], sem).wait()
        pltpu.make_async_copy(emb_hbm.at[idx_v.at[pl.ds(row * SP + C0, C1)]],
                              buf.at[pl.ds(C0, C1)], sem).wait()

    def reduce_row(row, buf):
        def acc_body(j8, accs):
            new = list(accs)
            for jj in range(ACC_UNROLL):
                j = j8 * ACC_UNROLL + jj
                for k in range(LANE_CHUNKS):
                    new[k] = new[k] + buf[j, pl.ds(k * L, L)]
            return tuple(new)

        accs = tuple(jnp.zeros((L,), jnp.float32) for _ in range(LANE_CHUNKS))
        accs = lax.fori_loop(0, S // ACC_UNROLL, acc_body, accs)
        for k in range(LANE_CHUNKS):
            out_v[row, pl.ds(k * L, L)] = accs[k]

    for p in range(NBUF):
        issue(p, bufs[p], sems[p])

    def body(blk, _):
        for p in range(NBUF):
            row = blk * NBUF + p
            wait(row, bufs[p], sems[p])
            reduce_row(row, bufs[p])
            issue(row + NBUF, bufs[p], sems[p])
        return ()

    lax.fori_loop(0, B_PER_W // NBUF - 1, body, ())

    for p in range(NBUF):
        row = B_PER_W - NBUF + p
        wait(row, bufs[p], sems[p])
        reduce_row(row, bufs[p])

    pltpu.sync_copy(out_v, out_hbm.at[pl.ds(base, B_PER_W)])


def _pad_idx_body(x_ref, o_ref):
    o_ref[:, :S] = x_ref[...]
    o_ref[:, S:] = jnp.zeros((B, SP - S), jnp.int32)


_pad_idx = pl.pallas_call(
    _pad_idx_body,
    out_shape=jax.ShapeDtypeStruct((B, SP), jnp.int32),
)


def _mlp_body(sums_ref, w1t_ref, b1_ref, w2t_ref, b2_ref, out_ref):
    pooled = sums_ref[...] * (1.0 / S)
    h = jnp.dot(pooled, w1t_ref[...], preferred_element_type=jnp.float32)
    h = jnp.maximum(h + b1_ref[...], 0.0)
    out_ref[...] = (
        jnp.dot(h, w2t_ref[...], preferred_element_type=jnp.float32)
        + b2_ref[...]
    )


_mlp = pl.pallas_call(
    _mlp_body,
    out_shape=jax.ShapeDtypeStruct((B, NCLS_PAD), jnp.float32),
)


def kernel(x, emb, W1, b1, W2, b2):
    x32 = x.astype(jnp.int32)
    x_flat = _pad_idx(x32).reshape(-1)
    sums = _pool_sums(x_flat, emb)
    w2t_pad = jnp.zeros((H, NCLS_PAD), jnp.float32).at[:, :NCLS].set(W2.T)
    b2_pad = jnp.zeros((1, NCLS_PAD), jnp.float32).at[0, :NCLS].set(b2)
    logits = _mlp(sums, W1.T, b1.reshape(1, H), w2t_pad, b2_pad)
    return logits[:, :NCLS]


# speedup vs baseline: 1.2263x; 1.0414x over previous
import functools

import jax
import jax.numpy as jnp
from jax import lax
from jax.experimental import pallas as pl
from jax.experimental.pallas import tpu as pltpu
from jax.experimental.pallas import tpu_sc as plsc

B = 4096
S = 200
D = 64
H = 64
NCLS = 4
NCLS_PAD = 8
VOCAB = 1000000

NC = 2
NS = 16
NW = NC * NS
L = 16
DP = 128

NBUF = 2
B_PER_W = B // NW
C0, C1 = 104, 96
LANE_CHUNKS = D // L
ACC_UNROLL = 8
SP = 256
N_IDX = B_PER_W * SP

VCHUNK = 8192

_mesh = plsc.VectorSubcoreMesh(core_axis_name="c", subcore_axis_name="s")


@functools.partial(
    pl.kernel,
    mesh=_mesh,
    compiler_params=pltpu.CompilerParams(use_tc_tiling_on_sc=False),
    out_type=jax.ShapeDtypeStruct((B, D), jnp.float32),
    scratch_types=[
        pltpu.VMEM((N_IDX,), jnp.int32),
        *[pltpu.VMEM((S, DP), jnp.float32) for _ in range(NBUF)],
        pltpu.VMEM((B_PER_W, D), jnp.float32),
        *[pltpu.SemaphoreType.DMA for _ in range(NBUF)],
    ],
)
def _pool_sums(x_hbm, emb_hbm, out_hbm, idx_v, *rest):
    bufs = rest[:NBUF]
    out_v = rest[NBUF]
    sems = rest[NBUF + 1 : NBUF + 1 + NBUF]

    wid = lax.axis_index("s") * NC + lax.axis_index("c")
    base = wid * B_PER_W

    pltpu.sync_copy(x_hbm.at[pl.ds(base * SP, N_IDX)], idx_v)

    def issue(row, buf, sem):
        pltpu.async_copy(emb_hbm.at[idx_v.at[pl.ds(row * SP, C0)]],
                         buf.at[pl.ds(0, C0)], sem)
        pltpu.async_copy(emb_hbm.at[idx_v.at[pl.ds(row * SP + C0, C1)]],
                         buf.at[pl.ds(C0, C1)], sem)

    def wait(row, buf, sem):
        pltpu.make_async_copy(emb_hbm.at[idx_v.at[pl.ds(row * SP, C0)]],
                              buf.at[pl.ds(0, C0)], sem).wait()
        pltpu.make_async_copy(emb_hbm.at[idx_v.at[pl.ds(row * SP + C0, C1)]],
                              buf.at[pl.ds(C0, C1)], sem).wait()

    def reduce_row(row, buf):
        def acc_body(j8, accs):
            new = list(accs)
            for jj in range(ACC_UNROLL):
                j = j8 * ACC_UNROLL + jj
                for k in range(LANE_CHUNKS):
                    new[k] = new[k] + buf[j, pl.ds(k * L, L)]
            return tuple(new)

        accs = tuple(jnp.zeros((L,), jnp.float32) for _ in range(LANE_CHUNKS))
        accs = lax.fori_loop(0, S // ACC_UNROLL, acc_body, accs)
        for k in range(LANE_CHUNKS):
            out_v[row, pl.ds(k * L, L)] = accs[k]

    for p in range(NBUF):
        issue(p, bufs[p], sems[p])

    def body(blk, _):
        for p in range(NBUF):
            row = blk * NBUF + p
            wait(row, bufs[p], sems[p])
            reduce_row(row, bufs[p])
            issue(row + NBUF, bufs[p], sems[p])
        return ()

    lax.fori_loop(0, B_PER_W // NBUF - 1, body, ())

    for p in range(NBUF):
        row = B_PER_W - NBUF + p
        wait(row, bufs[p], sems[p])
        reduce_row(row, bufs[p])

    pltpu.sync_copy(out_v, out_hbm.at[pl.ds(base, B_PER_W)])


def _repack_body(embt_ref, eye_ref, o_ref):
    t = lax.dot_general(embt_ref[...], eye_ref[...],
                        (((0,), (0,)), ((), ())),
                        precision=lax.Precision.HIGHEST,
                        preferred_element_type=jnp.float32)
    o_ref[:, :D] = t
    o_ref[:, D:] = jnp.zeros((VCHUNK, DP - D), jnp.float32)


_repack = pl.pallas_call(
    _repack_body,
    grid=((VOCAB + VCHUNK - 1) // VCHUNK,),
    in_specs=[
        pl.BlockSpec((D, VCHUNK), lambda i: (0, i)),
        pl.BlockSpec((D, D), lambda i: (0, 0)),
    ],
    out_specs=pl.BlockSpec((VCHUNK, DP), lambda i: (i, 0)),
    out_shape=jax.ShapeDtypeStruct((VOCAB, DP), jnp.float32),
)


def _pad_idx_body(x_ref, o_ref):
    o_ref[:, :S] = x_ref[...]
    o_ref[:, S:] = jnp.zeros((B, SP - S), jnp.int32)


_pad_idx = pl.pallas_call(
    _pad_idx_body,
    out_shape=jax.ShapeDtypeStruct((B, SP), jnp.int32),
)


def _mlp_body(sums_ref, w1t_ref, b1_ref, w2t_ref, b2_ref, out_ref):
    pooled = sums_ref[...] * (1.0 / S)
    h = jnp.dot(pooled, w1t_ref[...], preferred_element_type=jnp.float32)
    h = jnp.maximum(h + b1_ref[...], 0.0)
    out_ref[...] = (
        jnp.dot(h, w2t_ref[...], preferred_element_type=jnp.float32)
        + b2_ref[...]
    )


_mlp = pl.pallas_call(
    _mlp_body,
    out_shape=jax.ShapeDtypeStruct((B, NCLS_PAD), jnp.float32),
)


def kernel(x, emb, W1, b1, W2, b2):
    x32 = x.astype(jnp.int32)
    x_flat = _pad_idx(x32).reshape(-1)
    eye = jnp.eye(D, dtype=jnp.float32)
    table = _repack(emb.T, eye)
    sums = _pool_sums(x_flat, table)
    w2t_pad = jnp.zeros((H, NCLS_PAD), jnp.float32).at[:, :NCLS].set(W2.T)
    b2_pad = jnp.zeros((1, NCLS_PAD), jnp.float32).at[0, :NCLS].set(b2)
    logits = _mlp(sums, W1.T, b1.reshape(1, H), w2t_pad, b2_pad)
    return logits[:, :NCLS]


# speedup vs baseline: 2.3079x; 1.8820x over previous
import functools

import jax
import jax.numpy as jnp
from jax import lax
from jax.experimental import pallas as pl
from jax.experimental.pallas import tpu as pltpu
from jax.experimental.pallas import tpu_sc as plsc

B = 4096
S = 200
D = 64
H = 64
NCLS = 4
NCLS_PAD = 8
VOCAB = 1000000

NC = 2
NS = 16
NW = NC * NS
L = 16
DP = 128

NBUF = 4
B_PER_W = B // NW
C0, C1 = 104, 96
LANE_CHUNKS = D // L
ACC_UNROLL = 8
SP = 256
N_IDX = B_PER_W * SP

VCHUNK = 16384

_mesh = plsc.VectorSubcoreMesh(core_axis_name="c", subcore_axis_name="s")


@functools.partial(
    pl.kernel,
    mesh=_mesh,
    compiler_params=pltpu.CompilerParams(use_tc_tiling_on_sc=False),
    out_type=jax.ShapeDtypeStruct((B, D), jnp.float32),
    scratch_types=[
        pltpu.VMEM((N_IDX,), jnp.int32),
        *[pltpu.VMEM((S, D), jnp.float32) for _ in range(NBUF)],
        pltpu.VMEM((B_PER_W, D), jnp.float32),
        *[pltpu.SemaphoreType.DMA for _ in range(NBUF)],
    ],
)
def _pool_sums(x_hbm, emb_hbm, out_hbm, idx_v, *rest):
    bufs = rest[:NBUF]
    out_v = rest[NBUF]
    sems = rest[NBUF + 1 : NBUF + 1 + NBUF]

    wid = lax.axis_index("s") * NC + lax.axis_index("c")
    base = wid * B_PER_W

    pltpu.sync_copy(x_hbm.at[pl.ds(base * SP, N_IDX)], idx_v)

    def issue(row, buf, sem):
        pltpu.async_copy(emb_hbm.at[idx_v.at[pl.ds(row * SP, C0)]],
                         buf.at[pl.ds(0, C0)], sem)
        pltpu.async_copy(emb_hbm.at[idx_v.at[pl.ds(row * SP + C0, C1)]],
                         buf.at[pl.ds(C0, C1)], sem)

    def wait(row, buf, sem):
        pltpu.make_async_copy(emb_hbm.at[idx_v.at[pl.ds(row * SP, C0)]],
                              buf.at[pl.ds(0, C0)], sem).wait()
        pltpu.make_async_copy(emb_hbm.at[idx_v.at[pl.ds(row * SP + C0, C1)]],
                              buf.at[pl.ds(C0, C1)], sem).wait()

    def reduce_row(row, buf):
        def acc_body(j8, accs):
            new = list(accs)
            for jj in range(ACC_UNROLL):
                j = j8 * ACC_UNROLL + jj
                for k in range(LANE_CHUNKS):
                    new[k] = new[k] + buf[j, pl.ds(k * L, L)]
            return tuple(new)

        accs = tuple(jnp.zeros((L,), jnp.float32) for _ in range(LANE_CHUNKS))
        accs = lax.fori_loop(0, S // ACC_UNROLL, acc_body, accs)
        for k in range(LANE_CHUNKS):
            out_v[row, pl.ds(k * L, L)] = accs[k]

    for p in range(NBUF):
        issue(p, bufs[p], sems[p])

    def body(blk, _):
        for p in range(NBUF):
            row = blk * NBUF + p
            wait(row, bufs[p], sems[p])
            reduce_row(row, bufs[p])
            issue(row + NBUF, bufs[p], sems[p])
        return ()

    lax.fori_loop(0, B_PER_W // NBUF - 1, body, ())

    for p in range(NBUF):
        row = B_PER_W - NBUF + p
        wait(row, bufs[p], sems[p])
        reduce_row(row, bufs[p])

    pltpu.sync_copy(out_v, out_hbm.at[pl.ds(base, B_PER_W)])


def _repack_body(embt_ref, o_ref):
    o_ref[:, :D] = embt_ref[...].T
    o_ref[:, D:] = jnp.zeros((VCHUNK, DP - D), jnp.float32)


_repack = pl.pallas_call(
    _repack_body,
    grid=((VOCAB + VCHUNK - 1) // VCHUNK,),
    in_specs=[
        pl.BlockSpec((D, VCHUNK), lambda i: (0, i)),
    ],
    out_specs=pl.BlockSpec((VCHUNK, DP), lambda i: (i, 0)),
    out_shape=jax.ShapeDtypeStruct((VOCAB, DP), jnp.float32),
)


def _pad_idx_body(x_ref, o_ref):
    o_ref[:, :S] = x_ref[...] * 2
    o_ref[:, S:] = jnp.zeros((B, SP - S), jnp.int32)


_pad_idx = pl.pallas_call(
    _pad_idx_body,
    out_shape=jax.ShapeDtypeStruct((B, SP), jnp.int32),
)


def _mlp_body(sums_ref, w1t_ref, b1_ref, w2t_ref, b2_ref, out_ref):
    pooled = sums_ref[...] * (1.0 / S)
    h = jnp.dot(pooled, w1t_ref[...], preferred_element_type=jnp.float32)
    h = jnp.maximum(h + b1_ref[...], 0.0)
    out_ref[...] = (
        jnp.dot(h, w2t_ref[...], preferred_element_type=jnp.float32)
        + b2_ref[...]
    )


_mlp = pl.pallas_call(
    _mlp_body,
    out_shape=jax.ShapeDtypeStruct((B, NCLS_PAD), jnp.float32),
)


def kernel(x, emb, W1, b1, W2, b2):
    x32 = x.astype(jnp.int32)
    x_flat = _pad_idx(x32).reshape(-1)
    table = _repack(emb.T).reshape(2 * VOCAB, D)
    sums = _pool_sums(x_flat, table)
    w2t_pad = jnp.zeros((H, NCLS_PAD), jnp.float32).at[:, :NCLS].set(W2.T)
    b2_pad = jnp.zeros((1, NCLS_PAD), jnp.float32).at[0, :NCLS].set(b2)
    logits = _mlp(sums, W1.T, b1.reshape(1, H), w2t_pad, b2_pad)
    return logits[:, :NCLS]
